# baseline (speedup 1.0000x reference)
import functools

import jax
import jax.numpy as jnp
from jax import lax
from jax.experimental import pallas as pl
from jax.experimental.pallas import tpu as pltpu
from jax.experimental.pallas import tpu_sc as plsc

_L = 16


@functools.lru_cache(maxsize=None)
def _build(B, S, E, V):
    N = B * S
    NC, NS = 2, 16
    NW = NC * NS
    E2 = 2 * E
    per_w = N // NW
    assert N % NW == 0 and per_w % S == 0
    C = 2 * S
    NCH = per_w // C
    SUB = ((0, 104), (104, 104), (208, 104), (312, 88))
    assert sum(n for _, n in SUB) == C

    mesh = plsc.VectorSubcoreMesh(core_axis_name="c", subcore_axis_name="s")

    @functools.partial(
        pl.kernel,
        mesh=mesh,
        out_type=jax.ShapeDtypeStruct((N, E), jnp.float32),
        scratch_types=[
            pltpu.VMEM((per_w,), jnp.int32),
            pltpu.VMEM((per_w,), jnp.int32),
            pltpu.VMEM((C, E2), jnp.float32),
            pltpu.VMEM((C, E), jnp.float32),
            pltpu.VMEM((S * E,), jnp.float32),
            pltpu.SemaphoreType.DMA,
        ],
    )
    def body(x_hbm, tok_hbm, pos_hbm, out_hbm,
             idx_v, pidx_v, rows_v, rows2_v, pos_v, sem):
        wid = lax.axis_index("s") * NC + lax.axis_index("c")
        base = wid * per_w
        pltpu.sync_copy(x_hbm.at[pl.ds(base, per_w)], idx_v)
        pltpu.sync_copy(pos_hbm.at[pl.ds(0, S * E)], pos_v)

        def shift_body(i, carry):
            sl = pl.ds(i * _L, _L)
            pidx_v[sl] = lax.shift_right_logical(idx_v[sl], 1)
            return carry

        lax.fori_loop(0, per_w // _L, shift_body, 0)

        def chunk_body(g, carry):
            cbase = g * C
            copies = [
                pltpu.async_copy(
                    tok_hbm.at[pidx_v.at[pl.ds(cbase + off, n)]],
                    rows_v.at[pl.ds(off, n)],
                    sem,
                )
                for off, n in SUB
            ]
            for cp in copies:
                cp.wait()

            def grp_body(k, c2):
                rbase = k * _L
                ov = (idx_v[pl.ds(cbase + rbase, _L)] & 1) * E
                for j in range(_L):
                    rr = rbase + j
                    o = ov[j]
                    p = rr - S * (rr >= S).astype(jnp.int32)
                    for c4 in range(E // _L):
                        v = rows_v[rr, pl.ds(o + c4 * _L, _L)]
                        rows2_v[rr, pl.ds(c4 * _L, _L)] = (
                            v + pos_v[pl.ds(p * E + c4 * _L, _L)])
                return c2

            lax.fori_loop(0, C // _L, grp_body, 0)
            pltpu.sync_copy(rows2_v, out_hbm.at[pl.ds(base + cbase, C)])
            return carry

        lax.fori_loop(0, NCH, chunk_body, 0)

    return body


def kernel(x, tok_table, pos_table):
    B, S = x.shape
    V, E = tok_table.shape
    idx = x.astype(jnp.int32).reshape(-1)
    tok2 = tok_table.reshape(V // 2, 2 * E)
    out = _build(B, S, E, V)(idx, tok2, pos_table[:S].reshape(-1))
    return out.reshape(B, S, E)


# speedup vs baseline: 1.2856x; 1.2856x over previous
import functools

import jax
import jax.numpy as jnp
from jax import lax
from jax.experimental import pallas as pl
from jax.experimental.pallas import tpu as pltpu
from jax.experimental.pallas import tpu_sc as plsc

_LANES = 128


@functools.lru_cache(maxsize=None)
def _build(B, S, E2, V):
    N = B * S
    NC, NS = 2, 16
    NW = NC * NS
    assert N % NW == 0
    per_w = N // NW
    assert per_w % S == 0
    C = 2 * S
    NCH = per_w // C
    SUB = ((0, 104), (104, 104), (208, 104), (312, 88))
    assert sum(n for _, n in SUB) == C

    mesh = plsc.VectorSubcoreMesh(core_axis_name="c", subcore_axis_name="s")

    @functools.partial(
        pl.kernel,
        mesh=mesh,
        out_type=jax.ShapeDtypeStruct((N, E2), jnp.float32),
        scratch_types=[
            pltpu.VMEM((per_w,), jnp.int32),
            pltpu.VMEM((2 * C, E2), jnp.float32),
            pltpu.VMEM_SHARED((C, E2), jnp.float32),
            pltpu.SemaphoreType.DMA,
            pltpu.SemaphoreType.DMA,
        ],
    )
    def body(x_hbm, tok_hbm, pos_hbm, out_hbm, idx_v, rows_v, pos_sh, sem,
             wsem):
        sid = lax.axis_index("s")
        wid = sid * NC + lax.axis_index("c")
        base = wid * per_w
        pltpu.sync_copy(x_hbm.at[pl.ds(base, per_w)], idx_v)
        @pl.when(sid == 0)
        def _():
            pltpu.sync_copy(pos_hbm.at[pl.ds(0, S)], rows_v.at[pl.ds(0, S)])
            for h in range(C // S):
                pltpu.sync_copy(rows_v.at[pl.ds(0, S)],
                                pos_sh.at[pl.ds(h * S, S)])
        plsc.subcore_barrier()

        def chunk_body(g, carry):
            cbase = g * C
            boff = (g & 1) * C
            buf = rows_v.at[pl.ds(boff, C)]

            @pl.when(g >= 2)
            def _():
                pltpu.make_async_copy(
                    out_hbm.at[pl.ds(base, C)], buf, wsem).wait()

            pltpu.sync_copy(pos_sh, buf)
            copies = [
                pltpu.async_copy(
                    tok_hbm.at[idx_v.at[pl.ds(cbase + off, n)]],
                    buf.at[pl.ds(off, n)],
                    sem,
                    add=True,
                )
                for off, n in SUB
            ]
            for cp in copies:
                cp.wait()
            pltpu.async_copy(buf, out_hbm.at[pl.ds(base + cbase, C)], wsem)
            return carry

        lax.fori_loop(0, NCH, chunk_body, 0)
        for _ in range(2):
            pltpu.make_async_copy(
                out_hbm.at[pl.ds(base, C)], rows_v.at[pl.ds(0, C)],
                wsem).wait()

    return body


def kernel(x, tok_table, pos_table):
    B, S = x.shape
    V, E = tok_table.shape
    idx = x.astype(jnp.int32).reshape(-1)
    pad = _LANES - E
    tok128 = jnp.pad(tok_table, ((0, 0), (0, pad)))
    pos128 = jnp.pad(pos_table[:S], ((0, 0), (0, pad)))
    out = _build(B, S, _LANES, V)(idx, tok128, pos128)
    return out[:, :E].reshape(B, S, E)


# speedup vs baseline: 2.0710x; 1.6109x over previous
import functools

import jax
import jax.numpy as jnp
from jax import lax
from jax.experimental import pallas as pl
from jax.experimental.pallas import tpu as pltpu
from jax.experimental.pallas import tpu_sc as plsc

_LANES = 128


@functools.lru_cache(maxsize=None)
def _tc_pad_transpose(V, E):
    blk = 7936

    def body(in_ref, out_ref):
        t = jnp.transpose(in_ref[...], (1, 0))
        out_ref[:, :E] = t
        out_ref[:, E:] = jnp.zeros((blk, E), jnp.float32)

    return pl.pallas_call(
        body,
        grid=((V + blk - 1) // blk,),
        in_specs=[pl.BlockSpec((E, blk), lambda i: (0, i))],
        out_specs=pl.BlockSpec((blk, 2 * E), lambda i: (i, 0)),
        out_shape=jax.ShapeDtypeStruct((V, 2 * E), jnp.float32),
    )


@functools.lru_cache(maxsize=None)
def _build(B, S, E2, V):
    N = B * S
    NC, NS = 2, 16
    NW = NC * NS
    assert N % NW == 0
    per_w = N // NW
    assert per_w % S == 0
    C = 2 * S
    NCH = per_w // C
    SUB = ((0, 104), (104, 104), (208, 104), (312, 88))
    assert sum(n for _, n in SUB) == C

    mesh = plsc.VectorSubcoreMesh(core_axis_name="c", subcore_axis_name="s")

    @functools.partial(
        pl.kernel,
        mesh=mesh,
        out_type=jax.ShapeDtypeStruct((N, E2), jnp.float32),
        scratch_types=[
            pltpu.VMEM((per_w,), jnp.int32),
            pltpu.VMEM((2 * C, E2), jnp.float32),
            pltpu.VMEM_SHARED((C, E2), jnp.float32),
            pltpu.SemaphoreType.DMA,
            pltpu.SemaphoreType.DMA,
        ],
    )
    def body(x_hbm, tok_hbm, pos_hbm, out_hbm, idx_v, rows_v, pos_sh, sem,
             wsem):
        sid = lax.axis_index("s")
        wid = sid * NC + lax.axis_index("c")
        base = wid * per_w
        pltpu.sync_copy(x_hbm.at[pl.ds(base, per_w)], idx_v)
        @pl.when(sid == 0)
        def _():
            pltpu.sync_copy(pos_hbm.at[pl.ds(0, S)], rows_v.at[pl.ds(0, S)])
            for h in range(C // S):
                pltpu.sync_copy(rows_v.at[pl.ds(0, S)],
                                pos_sh.at[pl.ds(h * S, S)])
        plsc.subcore_barrier()

        def chunk_body(g, carry):
            cbase = g * C
            boff = (g & 1) * C
            buf = rows_v.at[pl.ds(boff, C)]

            @pl.when(g >= 2)
            def _():
                pltpu.make_async_copy(
                    out_hbm.at[pl.ds(base, C)], buf, wsem).wait()

            pltpu.sync_copy(pos_sh, buf)
            copies = [
                pltpu.async_copy(
                    tok_hbm.at[idx_v.at[pl.ds(cbase + off, n)]],
                    buf.at[pl.ds(off, n)],
                    sem,
                    add=True,
                )
                for off, n in SUB
            ]
            for cp in copies:
                cp.wait()
            pltpu.async_copy(buf, out_hbm.at[pl.ds(base + cbase, C)], wsem)
            return carry

        lax.fori_loop(0, NCH, chunk_body, 0)
        for _ in range(2):
            pltpu.make_async_copy(
                out_hbm.at[pl.ds(base, C)], rows_v.at[pl.ds(0, C)],
                wsem).wait()

    return body


def kernel(x, tok_table, pos_table):
    B, S = x.shape
    V, E = tok_table.shape
    idx = x.astype(jnp.int32).reshape(-1)
    pad = _LANES - E
    tok128 = _tc_pad_transpose(V, E)(tok_table.T)
    pos128 = jnp.pad(pos_table[:S], ((0, 0), (0, pad)))
    out = _build(B, S, _LANES, V)(idx, tok128, pos128)
    return out[:, :E].reshape(B, S, E)
